# Initial kernel scaffold; baseline (speedup 1.0000x reference)
import jax
import jax.numpy as jnp
from jax.experimental import pallas as pl


def kernel(n_feat, edge_index, edge_weights, W_in, b_in):
    raise NotImplementedError("write your pallas kernel here")


# speedup vs baseline: 5.0465x; 5.0465x over previous
import functools

import jax
import jax.numpy as jnp
from jax import lax
from jax.experimental import pallas as pl
from jax.experimental.pallas import tpu as pltpu
from jax.experimental.pallas import tpu_sc as plsc

N_NODES = 10000
D_FEAT = 128
N_EDGES = 320000

NC = 2
NS = 16
NW = NC * NS
CH = 128
NCH = 79
E_PAD = NW * NCH * CH
N_PAD = 10240
RPT = N_PAD // NS


def _sc_aggregate(n_feat, src, dst, w):
    mesh = plsc.VectorSubcoreMesh(core_axis_name="c", subcore_axis_name="s")

    @functools.partial(
        pl.kernel,
        mesh=mesh,
        out_type=jax.ShapeDtypeStruct((NC, N_PAD, D_FEAT), jnp.float32),
        scratch_types=[
            pltpu.VMEM_SHARED((N_PAD, D_FEAT), jnp.float32),
            pltpu.VMEM((NCH, CH), jnp.int32),
            pltpu.VMEM((NCH, CH), jnp.int32),
            pltpu.VMEM((NCH, CH), jnp.float32),
            pltpu.VMEM((CH, D_FEAT), jnp.float32),
            pltpu.SemaphoreType.DMA,
        ],
    )
    def body(nf_hbm, src_hbm, dst_hbm, w_hbm, out_hbm, acc, sidx, didx, wv,
             rows, sem):
        c = lax.axis_index("c")
        s = lax.axis_index("s")
        wid = c * NS + s

        pltpu.sync_copy(src_hbm.at[wid], sidx)
        pltpu.sync_copy(dst_hbm.at[wid], didx)
        pltpu.sync_copy(w_hbm.at[wid], wv)

        zero = jnp.zeros((16,), jnp.float32)

        def zrow(r, carry):
            for k in range(D_FEAT // 16):
                rows[r, pl.ds(k * 16, 16)] = zero
            return carry

        lax.fori_loop(0, CH, zrow, 0)
        for j in range(RPT // CH):
            pltpu.sync_copy(rows, acc.at[pl.ds(s * RPT + j * CH, CH)])
        plsc.subcore_barrier()

        def chunk(ci, carry):
            pltpu.async_copy(nf_hbm.at[sidx.at[ci]], rows, sem).wait()

            dnums = lax.GatherDimensionNumbers(
                offset_dims=(), collapsed_slice_dims=(0,),
                start_index_map=(0,))

            def grp(g, inner):
                w16 = wv[ci, pl.ds(g * 16, 16)]
                for j in range(16):
                    sp = lax.gather(
                        w16, jnp.full((16, 1), j, jnp.int32), dnums,
                        slice_sizes=(1,),
                        mode=lax.GatherScatterMode.PROMISE_IN_BOUNDS)
                    r = g * 16 + j
                    for k in range(D_FEAT // 16):
                        rows[r, pl.ds(k * 16, 16)] = (
                            rows[r, pl.ds(k * 16, 16)] * sp)
                return inner

            lax.fori_loop(0, CH // 16, grp, 0)
            pltpu.sync_copy(rows, acc.at[didx.at[ci]], add=True)
            return carry

        lax.fori_loop(0, NCH, chunk, 0)
        plsc.subcore_barrier()

        pltpu.sync_copy(acc.at[pl.ds(s * RPT, RPT)],
                        out_hbm.at[c, pl.ds(s * RPT, RPT)])

    return body(n_feat, src, dst, w)


def _tc_body(p_ref, m_ref, b_ref, o_ref):
    agg = p_ref[0] + p_ref[1]
    o_ref[...] = jnp.dot(agg, m_ref[...],
                         preferred_element_type=jnp.float32,
                         precision=lax.Precision.HIGHEST) + b_ref[...]


def kernel(n_feat, edge_index, edge_weights, W_in, b_in):
    src = edge_index[0].astype(jnp.int32)
    dst = edge_index[1].astype(jnp.int32)
    w = edge_weights.reshape(-1).astype(jnp.float32)

    pad = E_PAD - N_EDGES
    src = jnp.concatenate([src, jnp.zeros((pad,), jnp.int32)])
    dst = jnp.concatenate([dst, jnp.zeros((pad,), jnp.int32)])
    w = jnp.concatenate([w, jnp.zeros((pad,), jnp.float32)])
    src = src.reshape(NW, NCH, CH)
    dst = dst.reshape(NW, NCH, CH)
    w = w.reshape(NW, NCH, CH)

    partials = _sc_aggregate(n_feat, src, dst, w)[:, :N_NODES, :]

    m = W_in.T + jnp.eye(D_FEAT, dtype=jnp.float32)
    out = pl.pallas_call(
        _tc_body,
        out_shape=jax.ShapeDtypeStruct((N_NODES, D_FEAT), jnp.float32),
    )(partials, m, b_in.reshape(1, D_FEAT))
    return out
